# baseline (speedup 1.0000x reference)
import functools

import jax
import jax.numpy as jnp
from jax import lax
from jax.experimental import pallas as pl
from jax.experimental.pallas import tpu as pltpu
from jax.experimental.pallas import tpu_sc as plsc

H = 256


def _proj_body(x_ref, w_ref, b_ref, o_ref):
    o_ref[...] = lax.dot_general(
        x_ref[...], w_ref[...], (((1,), (1,)), ((), ()))) + b_ref[...]


def _project(x, w, b, bm):
    m, k = x.shape
    return pl.pallas_call(
        _proj_body,
        grid=(m // bm,),
        in_specs=[
            pl.BlockSpec((bm, k), lambda i: (i, 0)),
            pl.BlockSpec((H, k), lambda i: (0, 0)),
            pl.BlockSpec((1, H), lambda i: (0, 0)),
        ],
        out_specs=pl.BlockSpec((bm, H), lambda i: (i, 0)),
        out_shape=jax.ShapeDtypeStruct((m, H), jnp.float32),
    )(x, w, b.reshape(1, H))


@functools.lru_cache(maxsize=None)
def _make_seg(e_pad, npad):
    stripe = npad // 32
    mesh = plsc.VectorSubcoreMesh(core_axis_name="c", subcore_axis_name="s")

    @functools.partial(
        pl.kernel,
        mesh=mesh,
        compiler_params=pltpu.CompilerParams(needs_layout_passes=False),
        out_type=jax.ShapeDtypeStruct((npad, H), jnp.float32),
        scratch_types=[
            pltpu.VMEM((128,), jnp.int32),
            pltpu.VMEM((144,), jnp.int32),
            pltpu.VMEM((128,), jnp.int32),
            pltpu.VMEM((128, H), jnp.float32),
            pltpu.VMEM((128, H), jnp.float32),
            pltpu.VMEM((128, H), jnp.float32),
            pltpu.VMEM((40,), jnp.int32),
            pltpu.SemaphoreType.DMA,
        ],
    )
    def seg(h_hbm, src_hbm, dst_hbm, starts_hbm, zr_hbm, mean_hbm,
            srcv, dstv, dmod, stage, mstage, zrv, startsm, sem):
        cid = lax.axis_index("c")
        sid = lax.axis_index("s")
        wid = cid * 16 + sid
        pltpu.sync_copy(starts_hbm, startsm)
        pltpu.sync_copy(zr_hbm, zrv)

        for off in range(0, stripe, 128):
            sz = min(128, stripe - off)
            pltpu.sync_copy(zrv.at[pl.ds(0, sz)],
                            mean_hbm.at[pl.ds(wid * stripe + off, sz)])

        start = plsc.load_gather(
            startsm, [jnp.full((16,), wid, jnp.int32)])[0]
        hi = plsc.load_gather(
            startsm, [jnp.full((16,), wid + 1, jnp.int32)])[0]
        base = pl.multiple_of((start // 128) * 128, 128)
        nb = (hi - base + 127) // 128

        def batch(g, carry):
            prev, cnt, acc = carry
            eb = pl.multiple_of(base + g * 128, 128)
            pltpu.sync_copy(src_hbm.at[pl.ds(eb, 128)], srcv)
            pltpu.sync_copy(dst_hbm.at[pl.ds(eb, 144)], dstv)
            pltpu.async_copy(h_hbm.at[srcv], stage, sem).wait()
            for j in range(8):
                d16 = dstv[pl.ds(j * 16, 16)]
                dn16 = dstv[pl.ds(j * 16 + 1, 16)]
                eg = eb + j * 16 + lax.iota(jnp.int32, 16)
                keep = (eg >= start) & (eg < hi) & (d16 != dn16)
                dmod[pl.ds(j * 16, 16)] = jnp.where(
                    keep, d16, jnp.full((16,), npad - 1, jnp.int32))

            def chunk(c, ec):
                prev, cnt, acc = ec
                d16 = dstv[pl.ds(c * 16, 16)]
                for k in range(16):
                    e = c * 16 + k
                    d = d16[k]
                    sv = jnp.where(d == prev, 1.0, 0.0)
                    cnt = cnt * sv + 1.0
                    rm = jnp.ones((16,), jnp.float32) / cnt
                    new = []
                    for j in range(16):
                        s_j = stage[e, pl.ds(j * 16, 16)]
                        a_j = s_j + acc[j] * sv
                        mstage[e, pl.ds(j * 16, 16)] = a_j * rm
                        new.append(a_j)
                    acc = tuple(new)
                    prev = d
                return (prev, cnt, acc)

            carry = lax.fori_loop(0, 8, chunk, (prev, cnt, acc))
            pltpu.async_copy(mstage, mean_hbm.at[dmod], sem).wait()
            return carry

        zero16 = tuple(jnp.zeros((16,), jnp.float32) for _ in range(16))
        lax.fori_loop(0, nb, batch,
                      (jnp.int32(-1), jnp.zeros((16,), jnp.float32), zero16))

    return seg


def _segment_mean(h_src, edge, npad, zr):
    e = edge.shape[1]
    e_pad = -(-e // 4096) * 4096
    src = jnp.concatenate(
        [edge[0].astype(jnp.int32), jnp.zeros((e_pad - e,), jnp.int32)])
    dst = jnp.concatenate(
        [edge[1].astype(jnp.int32),
         jnp.full((e_pad - e + 256,), jnp.int32(1 << 30), jnp.int32)])
    order = jnp.argsort(dst)
    srcs = src[order]
    dsts = dst[order]
    stripe = npad // 32
    bounds = jnp.concatenate([
        jnp.arange(32, dtype=jnp.int32) * stripe,
        jnp.full((8,), npad - 1, jnp.int32),
    ])
    starts = jnp.searchsorted(dsts[:e_pad], bounds).astype(jnp.int32)
    return _make_seg(e_pad, npad)(h_src, srcs, dsts, starts, zr)


def _combine(h_dst, means, wls, wr_sum, bl_sum, bm=1000):
    n = h_dst.shape[0]
    r = len(means)

    def body(*refs):
        h_ref = refs[0]
        out_ref = refs[-1]
        wr_ref = refs[1 + 2 * r]
        b_ref = refs[2 + 2 * r]
        acc = lax.dot_general(
            h_ref[...], wr_ref[...], (((1,), (1,)), ((), ())))
        for j in range(r):
            mean = refs[1 + 2 * j][...]
            wl = refs[2 + 2 * j][...]
            acc += lax.dot_general(mean, wl, (((1,), (1,)), ((), ())))
        out_ref[...] = acc + b_ref[...]

    in_specs = [pl.BlockSpec((bm, H), lambda i: (i, 0))]
    args = [h_dst]
    for mean, wl in zip(means, wls):
        in_specs.append(pl.BlockSpec((bm, H), lambda i: (i, 0)))
        in_specs.append(pl.BlockSpec((H, H), lambda i: (0, 0)))
        args += [mean, wl]
    in_specs.append(pl.BlockSpec((H, H), lambda i: (0, 0)))
    in_specs.append(pl.BlockSpec((1, H), lambda i: (0, 0)))
    args += [wr_sum, bl_sum.reshape(1, H)]
    return pl.pallas_call(
        body,
        grid=(n // bm,),
        in_specs=in_specs,
        out_specs=pl.BlockSpec((bm, H), lambda i: (i, 0)),
        out_shape=jax.ShapeDtypeStruct((n, H), jnp.float32),
    )(*args)


def _round_npad(n):
    return -(-n // 256) * 256 + 256


def kernel(x_conversation, x_sentence, x_word, edge_cs, edge_ss, edge_sw,
           edge_ww, edge_sc, edge_ws, W_conv, b_conv, W_sent, b_sent,
           W_word, b_word, Wl_cs, bl_cs, Wr_cs, Wl_ss, bl_ss, Wr_ss,
           Wl_sw, bl_sw, Wr_sw, Wl_ww, bl_ww, Wr_ww, Wl_sc, bl_sc, Wr_sc,
           Wl_ws, bl_ws, Wr_ws):
    hc = _project(x_conversation, W_conv, b_conv, bm=1000)
    hs = _project(x_sentence, W_sent, b_sent, bm=1000)
    hw = _project(x_word, W_word, b_word, bm=1000)

    np_c = _round_npad(x_conversation.shape[0])
    np_s = _round_npad(x_sentence.shape[0])
    np_w = _round_npad(x_word.shape[0])

    zr = jnp.zeros((128, H), jnp.float32)

    m_cs = _segment_mean(hc, edge_cs, np_s, zr)
    m_ss = _segment_mean(hs, edge_ss, np_s, zr)
    m_ws = _segment_mean(hw, edge_ws, np_s, zr)
    m_sw = _segment_mean(hs, edge_sw, np_w, zr)
    m_ww = _segment_mean(hw, edge_ww, np_w, zr)
    m_sc = _segment_mean(hs, edge_sc, np_c, zr)

    out_s = _combine(hs, [m_cs, m_ss, m_ws], [Wl_cs, Wl_ss, Wl_ws],
                     Wr_cs + Wr_ss + Wr_ws, bl_cs + bl_ss + bl_ws)
    out_w = _combine(hw, [m_sw, m_ww], [Wl_sw, Wl_ww],
                     Wr_sw + Wr_ww, bl_sw + bl_ww)
    out_c = _combine(hc, [m_sc], [Wl_sc], Wr_sc, bl_sc)
    return (out_c, out_s, out_w)
